# baseline (speedup 1.0000x reference)
import dataclasses

import jax
import jax.numpy as jnp
from jax.experimental import pallas as pl
from jax.experimental.pallas import tpu as pltpu
from jax.experimental.pallas import tpu_sc as plsc

_BR = 400
_BR2 = 840
_SC_ROWS = 1600
_PAIRS_PER_SUB = 25
_SC_BRU = 400

_SC_CP = pltpu.CompilerParams()
if "needs_layout_passes" in pltpu.CompilerParams.__dataclass_fields__:
    _SC_CP = dataclasses.replace(_SC_CP, needs_layout_passes=False)


def _prep_kernel(a_ref, x_ref, w1t_ref, b1_ref, dis_ref, g1_ref, ab_ref):
    a = a_ref[...]
    deg = jnp.sum(a, axis=1, keepdims=True) + 1.0
    dis = jnp.where(deg > 0, jax.lax.rsqrt(deg), 0.0)
    h1 = (
        jnp.dot(x_ref[...], w1t_ref[...], preferred_element_type=jnp.float32)
        + b1_ref[...]
    )
    dis_ref[...] = dis
    g1_ref[...] = (dis * h1).astype(jnp.bfloat16)
    ab_ref[...] = a.astype(jnp.bfloat16)


def _sc_prep_kernel(
    a_ref, sums_ref, packed_ref, b0, b1, o0, o1, s0, s1, si0, si1, sp0, sp1, ss0, ss1
):
    n = a_ref.shape[1]
    nch = n // 16
    row0 = a_ref.shape[0] - _SC_ROWS
    core = jax.lax.axis_index("c")
    sub = jax.lax.axis_index("s")
    tbase = (core * 16 + sub) * _PAIRS_PER_SUB

    def in_copy(k, buf, sem):
        t = tbase + k
        return pltpu.make_async_copy(
            a_ref.at[pl.ds(row0 + 2 * t, 2), :], buf, sem
        )

    def out_copies(k, ob, sb, semp, sems):
        t = tbase + k
        return (
            pltpu.make_async_copy(ob, packed_ref.at[pl.ds(t, 1), :], semp),
            pltpu.make_async_copy(sb, sums_ref.at[pl.ds(2 * t, 2), :], sems),
        )

    def compute(buf, ob, sb):
        zero = jnp.zeros((16,), jnp.float32)

        def body(c, carry):
            a0, a1 = carry
            v0 = buf[0, pl.ds(c * 16, 16)]
            v1 = buf[1, pl.ds(c * 16, 16)]
            u0 = plsc.bitcast(v0, jnp.uint32)
            u1 = plsc.bitcast(v1, jnp.uint32)
            w = jax.lax.shift_right_logical(u0, jnp.uint32(16)) | (
                u1 & jnp.uint32(0xFFFF0000)
            )
            ob[0, pl.ds(c * 16, 16)] = w
            return (a0 + v0, a1 + v1)

        acc0, acc1 = jax.lax.fori_loop(0, nch, body, (zero, zero))
        sb[0, :] = acc0
        sb[1, :] = acc1

    bufs = ((b0, o0, s0, si0, sp0, ss0), (b1, o1, s1, si1, sp1, ss1))
    in_copy(0, b0, si0).start()
    in_copy(1, b1, si1).start()
    for k in range(_PAIRS_PER_SUB):
        buf, ob, sb, si, sp, ss = bufs[k % 2]
        if k >= 2:
            cp, cs = out_copies(k - 2, ob, sb, sp, ss)
            cp.wait()
            cs.wait()
        in_copy(k, buf, si).wait()
        compute(buf, ob, sb)
        cp, cs = out_copies(k, ob, sb, sp, ss)
        cp.start()
        cs.start()
        if k + 2 < _PAIRS_PER_SUB:
            in_copy(k + 2, buf, si).start()
    for k in (_PAIRS_PER_SUB - 2, _PAIRS_PER_SUB - 1):
        buf, ob, sb, si, sp, ss = bufs[k % 2]
        cp, cs = out_copies(k, ob, sb, sp, ss)
        cp.wait()
        cs.wait()


def _sc_finish_kernel(sums_ref, x_ref, w1t_ref, b1_ref, dis_ref, g1_ref):
    n = x_ref.shape[0]
    deg = jnp.sum(sums_ref[...], axis=1, keepdims=True) + 1.0
    dis = jnp.where(deg > 0, jax.lax.rsqrt(deg), 0.0)
    xs = x_ref[pl.ds(n - _SC_ROWS, _SC_ROWS), :]
    h1 = (
        jnp.dot(xs, w1t_ref[...], preferred_element_type=jnp.float32)
        + b1_ref[...]
    )
    dis_ref[...] = dis
    g1_ref[...] = (dis * h1).astype(jnp.bfloat16)


def _mid_kernel_tc(ab_ref, g1_ref, dis_ref, w2t_ref, b2_ref, g2_ref):
    i = pl.program_id(0)
    acc = jnp.dot(ab_ref[...], g1_ref[...], preferred_element_type=jnp.float32)
    g1_i = g1_ref[pl.ds(i * _BR2, _BR2), :].astype(jnp.float32)
    dis = dis_ref[...]
    out1 = jnp.maximum(dis * (acc + g1_i), 0.0)
    h2 = (
        jnp.dot(out1, w2t_ref[...], preferred_element_type=jnp.float32)
        + b2_ref[...]
    )
    g2_ref[...] = (dis * h2).astype(jnp.bfloat16)


def _mid_kernel_sc(p_ref, g1_ref, dis_ref, w2t_ref, b2_ref, g2_ref):
    i = pl.program_id(0)
    n = g1_ref.shape[0]
    ab = pltpu.bitcast(p_ref[...], jnp.bfloat16)
    acc = jnp.dot(ab, g1_ref[...], preferred_element_type=jnp.float32)
    g1_i = g1_ref[
        pl.ds(n - _SC_ROWS + i * 2 * _SC_BRU, 2 * _SC_BRU), :
    ].astype(jnp.float32)
    dis = dis_ref[...]
    out1 = jnp.maximum(dis * (acc + g1_i), 0.0)
    h2 = (
        jnp.dot(out1, w2t_ref[...], preferred_element_type=jnp.float32)
        + b2_ref[...]
    )
    g2_ref[...] = (dis * h2).astype(jnp.bfloat16)


def _final_kernel_tc(ab_ref, g2_ref, dis_ref, out_ref):
    i = pl.program_id(0)
    acc = jnp.dot(ab_ref[...], g2_ref[...], preferred_element_type=jnp.float32)
    g2_i = g2_ref[pl.ds(i * _BR2, _BR2), :].astype(jnp.float32)
    out_ref[...] = jnp.maximum(dis_ref[...] * (acc + g2_i), 0.0)


def _final_kernel_sc(p_ref, g2_ref, dis_ref, out_ref):
    i = pl.program_id(0)
    n = g2_ref.shape[0]
    ab = pltpu.bitcast(p_ref[...], jnp.bfloat16)
    acc = jnp.dot(ab, g2_ref[...], preferred_element_type=jnp.float32)
    g2_i = g2_ref[
        pl.ds(n - _SC_ROWS + i * 2 * _SC_BRU, 2 * _SC_BRU), :
    ].astype(jnp.float32)
    out_ref[...] = jnp.maximum(dis_ref[...] * (acc + g2_i), 0.0)


@jax.jit
def kernel(x, graph_structure, W1, b1, W2, b2):
    n, d_in = x.shape
    hid = W1.shape[0]
    out_dim = W2.shape[0]
    a = graph_structure
    w1t = W1.T
    w2t = W2.T
    b1r = b1.reshape(1, hid)
    b2r = b2.reshape(1, out_dim)

    n_tc = n - _SC_ROWS

    a_spec = pl.BlockSpec((_BR, n), lambda i: (i, 0))
    row_vec = lambda w: pl.BlockSpec((_BR, w), lambda i: (i, 0))
    full = lambda s: pl.BlockSpec(s, lambda i: (0, 0))

    dis_tc, g1_tc, ab_tc = pl.pallas_call(
        _prep_kernel,
        grid=(n_tc // _BR,),
        in_specs=[a_spec, row_vec(d_in), full((d_in, hid)), full((1, hid))],
        out_specs=[row_vec(1), row_vec(hid), a_spec],
        out_shape=[
            jax.ShapeDtypeStruct((n_tc, 1), jnp.float32),
            jax.ShapeDtypeStruct((n_tc, hid), jnp.bfloat16),
            jax.ShapeDtypeStruct((n_tc, n), jnp.bfloat16),
        ],
    )(a, x, w1t, b1r)

    @pl.kernel(
        out_type=[
            jax.ShapeDtypeStruct((_SC_ROWS, 16), jnp.float32),
            jax.ShapeDtypeStruct((_SC_ROWS // 2, n), jnp.uint32),
        ],
        mesh=plsc.VectorSubcoreMesh(core_axis_name="c", subcore_axis_name="s"),
        compiler_params=_SC_CP,
        scratch_types=[
            pltpu.VMEM((2, n), jnp.float32),
            pltpu.VMEM((2, n), jnp.float32),
            pltpu.VMEM((1, n), jnp.uint32),
            pltpu.VMEM((1, n), jnp.uint32),
            pltpu.VMEM((2, 16), jnp.float32),
            pltpu.VMEM((2, 16), jnp.float32),
            pltpu.SemaphoreType.DMA,
            pltpu.SemaphoreType.DMA,
            pltpu.SemaphoreType.DMA,
            pltpu.SemaphoreType.DMA,
            pltpu.SemaphoreType.DMA,
            pltpu.SemaphoreType.DMA,
        ],
    )
    def sc_part(a_ref, sums_ref, packed_ref, *scratch):
        _sc_prep_kernel(a_ref, sums_ref, packed_ref, *scratch)

    sums_sc, packed_sc = sc_part(a)

    dis_sc, g1_sc = pl.pallas_call(
        _sc_finish_kernel,
        grid=(1,),
        in_specs=[
            full((_SC_ROWS, 16)),
            full((n, d_in)),
            full((d_in, hid)),
            full((1, hid)),
        ],
        out_specs=[full((_SC_ROWS, 1)), full((_SC_ROWS, hid))],
        out_shape=[
            jax.ShapeDtypeStruct((_SC_ROWS, 1), jnp.float32),
            jax.ShapeDtypeStruct((_SC_ROWS, hid), jnp.bfloat16),
        ],
    )(sums_sc, x, w1t, b1r)

    g1 = jnp.concatenate([g1_tc, g1_sc], axis=0)

    ab2_spec = pl.BlockSpec((_BR2, n), lambda i: (i, 0))
    row2 = lambda w: pl.BlockSpec((_BR2, w), lambda i: (i, 0))
    p_spec = pl.BlockSpec((_SC_BRU, n), lambda i: (i, 0))
    rowp = lambda w: pl.BlockSpec((2 * _SC_BRU, w), lambda i: (i, 0))

    g2_tc = pl.pallas_call(
        _mid_kernel_tc,
        grid=(n_tc // _BR2,),
        in_specs=[
            ab2_spec,
            full((n, hid)),
            row2(1),
            full((hid, out_dim)),
            full((1, out_dim)),
        ],
        out_specs=row2(out_dim),
        out_shape=jax.ShapeDtypeStruct((n_tc, out_dim), jnp.bfloat16),
    )(ab_tc, g1, dis_tc, w2t, b2r)

    g2_sc = pl.pallas_call(
        _mid_kernel_sc,
        grid=(_SC_ROWS // (2 * _SC_BRU),),
        in_specs=[
            p_spec,
            full((n, hid)),
            rowp(1),
            full((hid, out_dim)),
            full((1, out_dim)),
        ],
        out_specs=rowp(out_dim),
        out_shape=jax.ShapeDtypeStruct((_SC_ROWS, out_dim), jnp.bfloat16),
    )(packed_sc, g1, dis_sc, w2t, b2r)

    g2 = jnp.concatenate([g2_tc, g2_sc], axis=0)

    out_tc = pl.pallas_call(
        _final_kernel_tc,
        grid=(n_tc // _BR2,),
        in_specs=[ab2_spec, full((n, out_dim)), row2(1)],
        out_specs=row2(out_dim),
        out_shape=jax.ShapeDtypeStruct((n_tc, out_dim), jnp.float32),
    )(ab_tc, g2, dis_tc)

    out_sc = pl.pallas_call(
        _final_kernel_sc,
        grid=(_SC_ROWS // (2 * _SC_BRU),),
        in_specs=[p_spec, full((n, out_dim)), rowp(1)],
        out_specs=rowp(out_dim),
        out_shape=jax.ShapeDtypeStruct((_SC_ROWS, out_dim), jnp.float32),
    )(packed_sc, g2, dis_sc)

    return jnp.concatenate([out_tc, out_sc], axis=0)


# speedup vs baseline: 1.1529x; 1.1529x over previous
import jax
import jax.numpy as jnp
from jax.experimental import pallas as pl

_BR = 400
_BR2 = 1000


def _prep_kernel(a_ref, x_ref, w1t_ref, b1_ref, dis_ref, g1_ref, ab_ref):
    a = a_ref[...]
    deg = jnp.sum(a, axis=1, keepdims=True) + 1.0
    dis = jnp.where(deg > 0, jax.lax.rsqrt(deg), 0.0)
    h1 = (
        jnp.dot(x_ref[...], w1t_ref[...], preferred_element_type=jnp.float32)
        + b1_ref[...]
    )
    dis_ref[...] = dis
    g1_ref[...] = (dis * h1).astype(jnp.bfloat16)
    ab_ref[...] = a.astype(jnp.bfloat16)


_NS = 5
_HB = _BR2 // _NS


def _mid_kernel(*refs):
    ab_refs = refs[:_NS]
    g1_ref, dis_ref, w2t_ref, b2_ref, g2_ref = refs[_NS:]
    i = pl.program_id(0)
    g1 = g1_ref[...]
    dis = dis_ref[...]
    for h, ab_ref in enumerate(ab_refs):
        acc = jnp.dot(ab_ref[...], g1, preferred_element_type=jnp.float32)
        g1_i = g1_ref[pl.ds(i * _BR2 + h * _HB, _HB), :].astype(jnp.float32)
        dis_h = dis[h * _HB:(h + 1) * _HB, :]
        out1 = jnp.maximum(dis_h * (acc + g1_i), 0.0)
        h2 = (
            jnp.dot(out1, w2t_ref[...], preferred_element_type=jnp.float32)
            + b2_ref[...]
        )
        g2_ref[h * _HB:(h + 1) * _HB, :] = (dis_h * h2).astype(jnp.bfloat16)


def _final_kernel(*refs):
    ab_refs = refs[:_NS]
    g2_ref, dis_ref, out_ref = refs[_NS:]
    i = pl.program_id(0)
    g2 = g2_ref[...]
    dis = dis_ref[...]
    for h, ab_ref in enumerate(ab_refs):
        acc = jnp.dot(ab_ref[...], g2, preferred_element_type=jnp.float32)
        g2_i = g2_ref[pl.ds(i * _BR2 + h * _HB, _HB), :].astype(jnp.float32)
        dis_h = dis[h * _HB:(h + 1) * _HB, :]
        out_ref[h * _HB:(h + 1) * _HB, :] = jnp.maximum(
            dis_h * (acc + g2_i), 0.0
        )


@jax.jit
def kernel(x, graph_structure, W1, b1, W2, b2):
    n, d_in = x.shape
    hid = W1.shape[0]
    out_dim = W2.shape[0]
    a = graph_structure
    w1t = W1.T
    w2t = W2.T
    b1r = b1.reshape(1, hid)
    b2r = b2.reshape(1, out_dim)
    grid = (n // _BR,)

    a_spec = pl.BlockSpec((_BR, n), lambda i: (i, 0))
    a2_specs = [
        pl.BlockSpec((_HB, n), lambda i, k=k: (_NS * i + k, 0))
        for k in range(_NS)
    ]
    row_vec = lambda w: pl.BlockSpec((_BR, w), lambda i: (i, 0))
    row_vec2 = lambda w: pl.BlockSpec((_BR2, w), lambda i: (i, 0))
    full = lambda s: pl.BlockSpec(s, lambda i: (0, 0))

    dis, g1, ab = pl.pallas_call(
        _prep_kernel,
        grid=grid,
        in_specs=[a_spec, row_vec(d_in), full((d_in, hid)), full((1, hid))],
        out_specs=[row_vec(1), row_vec(hid), a_spec],
        out_shape=[
            jax.ShapeDtypeStruct((n, 1), jnp.float32),
            jax.ShapeDtypeStruct((n, hid), jnp.bfloat16),
            jax.ShapeDtypeStruct((n, n), jnp.bfloat16),
        ],
    )(a, x, w1t, b1r)

    grid2 = (n // _BR2,)
    g2 = pl.pallas_call(
        _mid_kernel,
        grid=grid2,
        in_specs=a2_specs
        + [
            full((n, hid)),
            row_vec2(1),
            full((hid, out_dim)),
            full((1, out_dim)),
        ],
        out_specs=row_vec2(out_dim),
        out_shape=jax.ShapeDtypeStruct((n, out_dim), jnp.bfloat16),
    )(*([ab] * _NS), g1, dis, w2t, b2r)

    out = pl.pallas_call(
        _final_kernel,
        grid=grid2,
        in_specs=a2_specs + [full((n, out_dim)), row_vec2(1)],
        out_specs=row_vec2(out_dim),
        out_shape=jax.ShapeDtypeStruct((n, out_dim), jnp.float32),
    )(*([ab] * _NS), g2, dis)

    return out
